# baseline (speedup 1.0000x reference)
import jax
import jax.numpy as jnp
from jax.experimental import pallas as pl
from jax.experimental.pallas import tpu as pltpu

COMMITMENT_COST = 0.25
BN = 256
KWIN = 4096


def _vq_body(zb_ref, z2_ref, w2_ref, wt_ref, w_ref,
             zq_ref, idx_ref, comm_ref, vq_ref, acc_ref):
    i = pl.program_id(0)
    nk = wt_ref.shape[1]
    zb = zb_ref[...]
    m = jax.lax.dot_general(zb, wt_ref[...], (((1,), (0,)), ((), ())),
                            preferred_element_type=jnp.float32)
    dist = (z2_ref[...] + w2_ref[...]) - 2.0 * m

    acc_v = None
    acc_i = None
    for w in range(nk // KWIN):
        dw = dist[:, w * KWIN:(w + 1) * KWIN]
        mv = jnp.min(dw, axis=1, keepdims=True)
        iw = jax.lax.broadcasted_iota(jnp.int32, dw.shape, 1) + w * KWIN
        ix = jnp.min(jnp.where(dw == mv, iw, nk), axis=1, keepdims=True)
        mv_bf = mv.astype(jnp.bfloat16).astype(jnp.float32)
        if acc_v is None:
            acc_v, acc_i = mv_bf, ix
        else:
            upd = mv < acc_v
            acc_i = jnp.where(upd, ix, acc_i)
            acc_v = jnp.where(upd, mv_bf, acc_v)
    idx = acc_i

    iota = jax.lax.broadcasted_iota(jnp.int32, dist.shape, 1)
    onehot = (iota == idx).astype(jnp.float32)
    zq = jax.lax.dot_general(onehot, w_ref[...], (((1,), (0,)), ((), ())),
                             preferred_element_type=jnp.float32)
    zq_ref[...] = zb + (zq - zb)
    idx_ref[...] = idx

    d = zb - zq
    part = jnp.sum(d * d)

    @pl.when(i == 0)
    def _init():
        acc_ref[0] = 0.0

    acc_ref[0] += part

    @pl.when(i == pl.num_programs(0) - 1)
    def _fin():
        c = acc_ref[0] / (zq_ref.shape[0] * pl.num_programs(0) * zq_ref.shape[1])
        comm_ref[0, 0] = c
        vq_ref[0, 0] = COMMITMENT_COST * c


def kernel(z_e, codebook):
    n, d = z_e.shape
    k = codebook.shape[0]
    z2 = jnp.sum(z_e ** 2, axis=1, keepdims=True)
    w2 = jnp.sum(codebook ** 2, axis=1)[None, :]
    wt = codebook.T

    grid = (n // BN,)
    zq_st, idx, comm, vq = pl.pallas_call(
        _vq_body,
        grid=grid,
        in_specs=[
            pl.BlockSpec((BN, d), lambda i: (i, 0)),
            pl.BlockSpec((BN, 1), lambda i: (i, 0)),
            pl.BlockSpec((1, k), lambda i: (0, 0)),
            pl.BlockSpec((d, k), lambda i: (0, 0)),
            pl.BlockSpec((k, d), lambda i: (0, 0)),
        ],
        out_specs=[
            pl.BlockSpec((BN, d), lambda i: (i, 0)),
            pl.BlockSpec((BN, 1), lambda i: (i, 0)),
            pl.BlockSpec(memory_space=pltpu.SMEM),
            pl.BlockSpec(memory_space=pltpu.SMEM),
        ],
        out_shape=[
            jax.ShapeDtypeStruct((n, d), jnp.float32),
            jax.ShapeDtypeStruct((n, 1), jnp.int32),
            jax.ShapeDtypeStruct((1, 1), jnp.float32),
            jax.ShapeDtypeStruct((1, 1), jnp.float32),
        ],
        scratch_shapes=[pltpu.SMEM((1,), jnp.float32)],
    )(z_e, z2, w2, wt, codebook)

    return (zq_st,
            jnp.reshape(vq, ()),
            jnp.reshape(comm, ()),
            jnp.reshape(idx, (n,)))


# speedup vs baseline: 1.6743x; 1.6743x over previous
import functools

import jax
import jax.numpy as jnp
from jax import lax
from jax.experimental import pallas as pl
from jax.experimental.pallas import tpu as pltpu
from jax.experimental.pallas import tpu_sc as plsc

COMMITMENT_COST = 0.25
BN = 256
KWIN = 4096

_SC_CORES = 2
_SC_SUBCORES = 16
_SC_WORKERS = _SC_CORES * _SC_SUBCORES


def _vq_body(zb_ref, z2_ref, w2_ref, wt2_ref,
             idx_ref, comm_ref, vq_ref, acc_ref):
    i = pl.program_id(0)
    nk = wt2_ref.shape[1]
    zb = zb_ref[...]
    m2 = jax.lax.dot_general(zb, wt2_ref[...], (((1,), (0,)), ((), ())),
                             preferred_element_type=jnp.float32)
    dist = (z2_ref[...] + w2_ref[...]) - m2

    acc_v = None
    acc_i = None
    acc_d = None
    for w in range(nk // KWIN):
        dw = dist[:, w * KWIN:(w + 1) * KWIN]
        mv = jnp.min(dw, axis=1, keepdims=True)
        iw = jax.lax.broadcasted_iota(jnp.int32, dw.shape, 1) + w * KWIN
        ix = jnp.min(jnp.where(dw == mv, iw, nk), axis=1, keepdims=True)
        mv_bf = mv.astype(jnp.bfloat16).astype(jnp.float32)
        if acc_v is None:
            acc_v, acc_i, acc_d = mv_bf, ix, mv
        else:
            upd = mv < acc_v
            acc_i = jnp.where(upd, ix, acc_i)
            acc_d = jnp.where(upd, mv, acc_d)
            acc_v = jnp.where(upd, mv_bf, acc_v)
    idx_ref[...] = acc_i

    part = jnp.sum(acc_d)

    @pl.when(i == 0)
    def _init():
        acc_ref[0] = 0.0

    acc_ref[0] += part

    @pl.when(i == pl.num_programs(0) - 1)
    def _fin():
        c = acc_ref[0] / (idx_ref.shape[0] * pl.num_programs(0) * zb_ref.shape[1])
        comm_ref[0, 0] = c
        vq_ref[0, 0] = COMMITMENT_COST * c


_SC_LANES = 128


def _make_sc_gather(k, n):
    b_per_w = n // _SC_WORKERS
    mesh = plsc.VectorSubcoreMesh(core_axis_name="c", subcore_axis_name="s")

    @functools.partial(
        pl.kernel, mesh=mesh,
        out_type=jax.ShapeDtypeStruct((n, _SC_LANES), jnp.float32),
        scratch_types=[
            pltpu.VMEM((b_per_w,), jnp.int32),
            pltpu.VMEM((b_per_w, _SC_LANES), jnp.float32),
            pltpu.SemaphoreType.DMA,
        ],
    )
    def gather_kernel(table_hbm, idx_hbm, out_hbm, idx_v, rows_v, sem):
        wid = lax.axis_index("s") * _SC_CORES + lax.axis_index("c")
        base = wid * b_per_w
        pltpu.sync_copy(idx_hbm.at[pl.ds(base, b_per_w)], idx_v)
        pltpu.async_copy(table_hbm.at[idx_v], rows_v, sem).wait()
        pltpu.sync_copy(rows_v, out_hbm.at[pl.ds(base, b_per_w)])

    return gather_kernel


def kernel(z_e, codebook):
    n, d = z_e.shape
    k = codebook.shape[0]
    z2 = jnp.sum(z_e ** 2, axis=1, keepdims=True)
    w2 = jnp.sum(codebook ** 2, axis=1)[None, :]
    wt2 = (2.0 * codebook).T

    grid = (n // BN,)
    idx, comm, vq = pl.pallas_call(
        _vq_body,
        grid=grid,
        in_specs=[
            pl.BlockSpec((BN, d), lambda i: (i, 0)),
            pl.BlockSpec((BN, 1), lambda i: (i, 0)),
            pl.BlockSpec((1, k), lambda i: (0, 0)),
            pl.BlockSpec((d, k), lambda i: (0, 0)),
        ],
        out_specs=[
            pl.BlockSpec((BN, 1), lambda i: (i, 0)),
            pl.BlockSpec(memory_space=pltpu.SMEM),
            pl.BlockSpec(memory_space=pltpu.SMEM),
        ],
        out_shape=[
            jax.ShapeDtypeStruct((n, 1), jnp.int32),
            jax.ShapeDtypeStruct((1, 1), jnp.float32),
            jax.ShapeDtypeStruct((1, 1), jnp.float32),
        ],
        scratch_shapes=[pltpu.SMEM((1,), jnp.float32)],
    )(z_e, z2, w2, wt2)

    idx_flat = jnp.reshape(idx, (n,))
    cb_pad = jnp.pad(codebook, ((0, 0), (0, _SC_LANES - d)))
    zq_st = _make_sc_gather(k, n)(cb_pad, idx_flat)[:, :d]

    return (zq_st,
            jnp.reshape(vq, ()),
            jnp.reshape(comm, ()),
            idx_flat)


# speedup vs baseline: 1.6779x; 1.0021x over previous
import functools

import jax
import jax.numpy as jnp
from jax import lax
from jax.experimental import pallas as pl
from jax.experimental.pallas import tpu as pltpu
from jax.experimental.pallas import tpu_sc as plsc

COMMITMENT_COST = 0.25
BN = 256
KWIN = 4096

_SC_CORES = 2
_SC_SUBCORES = 16
_SC_WORKERS = _SC_CORES * _SC_SUBCORES


def _vq_body(zb_ref, z2_ref, w2_ref, wt2_ref,
             idx_ref, comm_ref, vq_ref, acc_ref):
    i = pl.program_id(0)
    nk = wt2_ref.shape[1]
    zb = zb_ref[...]
    m2 = jax.lax.dot_general(zb, wt2_ref[...], (((1,), (0,)), ((), ())),
                             preferred_element_type=jnp.float32)
    dist = (z2_ref[...] + w2_ref[...]) - m2

    acc_v = None
    acc_i = None
    acc_d = None
    iota_w = jax.lax.broadcasted_iota(jnp.int32, (zb.shape[0], KWIN), 1)
    for w in range(nk // KWIN):
        dw = dist[:, w * KWIN:(w + 1) * KWIN]
        mv = jnp.min(dw, axis=1, keepdims=True)
        ix = jnp.min(jnp.where(dw == mv, iota_w, KWIN),
                     axis=1, keepdims=True) + w * KWIN
        mv_bf = mv.astype(jnp.bfloat16).astype(jnp.float32)
        if acc_v is None:
            acc_v, acc_i, acc_d = mv_bf, ix, mv
        else:
            upd = mv < acc_v
            acc_i = jnp.where(upd, ix, acc_i)
            acc_d = jnp.where(upd, mv, acc_d)
            acc_v = jnp.where(upd, mv_bf, acc_v)
    idx_ref[...] = acc_i

    part = jnp.sum(acc_d)

    @pl.when(i == 0)
    def _init():
        acc_ref[0] = 0.0

    acc_ref[0] += part

    @pl.when(i == pl.num_programs(0) - 1)
    def _fin():
        c = acc_ref[0] / (idx_ref.shape[0] * pl.num_programs(0) * zb_ref.shape[1])
        comm_ref[0, 0] = c
        vq_ref[0, 0] = COMMITMENT_COST * c


_SC_LANES = 128


def _make_sc_gather(k, n, d):
    b_per_w = n // _SC_WORKERS
    mesh = plsc.VectorSubcoreMesh(core_axis_name="c", subcore_axis_name="s")

    @functools.partial(
        pl.kernel, mesh=mesh,
        out_type=jax.ShapeDtypeStruct((n, _SC_LANES), jnp.float32),
        scratch_types=[
            pltpu.VMEM((b_per_w,), jnp.int32),
            pltpu.VMEM((b_per_w, _SC_LANES), jnp.float32),
            pltpu.SemaphoreType.DMA,
        ],
    )
    def gather_kernel(table_hbm, idx_hbm, out_hbm, idx_v, rows_v, sem):
        wid = lax.axis_index("s") * _SC_CORES + lax.axis_index("c")
        base = wid * b_per_w
        pltpu.sync_copy(idx_hbm.at[pl.ds(base, b_per_w)], idx_v)
        pltpu.async_copy(table_hbm.at[idx_v], rows_v, sem).wait()
        pltpu.sync_copy(rows_v, out_hbm.at[pl.ds(base, b_per_w)])

    return gather_kernel


def kernel(z_e, codebook):
    n, d = z_e.shape
    k = codebook.shape[0]
    z2 = jnp.sum(z_e ** 2, axis=1, keepdims=True)
    w2 = jnp.sum(codebook ** 2, axis=1)[None, :]
    wt2 = (2.0 * codebook).T

    grid = (n // BN,)
    idx, comm, vq = pl.pallas_call(
        _vq_body,
        grid=grid,
        in_specs=[
            pl.BlockSpec((BN, d), lambda i: (i, 0)),
            pl.BlockSpec((BN, 1), lambda i: (i, 0)),
            pl.BlockSpec((1, k), lambda i: (0, 0)),
            pl.BlockSpec((d, k), lambda i: (0, 0)),
        ],
        out_specs=[
            pl.BlockSpec((BN, 1), lambda i: (i, 0)),
            pl.BlockSpec(memory_space=pltpu.SMEM),
            pl.BlockSpec(memory_space=pltpu.SMEM),
        ],
        out_shape=[
            jax.ShapeDtypeStruct((n, 1), jnp.int32),
            jax.ShapeDtypeStruct((1, 1), jnp.float32),
            jax.ShapeDtypeStruct((1, 1), jnp.float32),
        ],
        scratch_shapes=[pltpu.SMEM((1,), jnp.float32)],
    )(z_e, z2, w2, wt2)

    idx_flat = jnp.reshape(idx, (n,))
    cb_pad = jnp.pad(codebook, ((0, 0), (0, _SC_LANES - d)))
    zq_st = _make_sc_gather(k, n, d)(cb_pad, idx_flat)[:, :d]

    return (zq_st,
            jnp.reshape(vq, ()),
            jnp.reshape(comm, ()),
            idx_flat)


# speedup vs baseline: 1.7159x; 1.0226x over previous
import functools

import jax
import jax.numpy as jnp
from jax import lax
from jax.experimental import pallas as pl
from jax.experimental.pallas import tpu as pltpu
from jax.experimental.pallas import tpu_sc as plsc

COMMITMENT_COST = 0.25
BN = 512
KWIN = 4096

_SC_CORES = 2
_SC_SUBCORES = 16
_SC_WORKERS = _SC_CORES * _SC_SUBCORES


def _vq_body(zb_ref, z2_ref, w2_ref, wt2_ref,
             idx_ref, comm_ref, vq_ref, acc_ref):
    i = pl.program_id(0)
    nk = wt2_ref.shape[1]
    zb = zb_ref[...]
    m2 = jax.lax.dot_general(zb, wt2_ref[...], (((1,), (0,)), ((), ())),
                             preferred_element_type=jnp.float32)
    dist = (z2_ref[...] + w2_ref[...]) - m2

    acc_v = None
    acc_i = None
    acc_d = None
    iota_w = jax.lax.broadcasted_iota(jnp.int32, (zb.shape[0], KWIN), 1)
    for w in range(nk // KWIN):
        dw = dist[:, w * KWIN:(w + 1) * KWIN]
        mv = jnp.min(dw, axis=1, keepdims=True)
        ix = jnp.min(jnp.where(dw == mv, iota_w, KWIN),
                     axis=1, keepdims=True) + w * KWIN
        mv_bf = mv.astype(jnp.bfloat16).astype(jnp.float32)
        if acc_v is None:
            acc_v, acc_i, acc_d = mv_bf, ix, mv
        else:
            upd = mv < acc_v
            acc_i = jnp.where(upd, ix, acc_i)
            acc_d = jnp.where(upd, mv, acc_d)
            acc_v = jnp.where(upd, mv_bf, acc_v)
    idx_ref[...] = acc_i

    part = jnp.sum(acc_d)

    @pl.when(i == 0)
    def _init():
        acc_ref[0] = 0.0

    acc_ref[0] += part

    @pl.when(i == pl.num_programs(0) - 1)
    def _fin():
        c = acc_ref[0] / (idx_ref.shape[0] * pl.num_programs(0) * zb_ref.shape[1])
        comm_ref[0, 0] = c
        vq_ref[0, 0] = COMMITMENT_COST * c


_SC_LANES = 128


def _make_sc_gather(k, n, d):
    b_per_w = n // _SC_WORKERS
    mesh = plsc.VectorSubcoreMesh(core_axis_name="c", subcore_axis_name="s")

    @functools.partial(
        pl.kernel, mesh=mesh,
        out_type=jax.ShapeDtypeStruct((n, _SC_LANES), jnp.float32),
        scratch_types=[
            pltpu.VMEM((b_per_w,), jnp.int32),
            pltpu.VMEM((b_per_w, _SC_LANES), jnp.float32),
            pltpu.SemaphoreType.DMA,
        ],
    )
    def gather_kernel(table_hbm, idx_hbm, out_hbm, idx_v, rows_v, sem):
        wid = lax.axis_index("s") * _SC_CORES + lax.axis_index("c")
        base = wid * b_per_w
        pltpu.sync_copy(idx_hbm.at[pl.ds(base, b_per_w)], idx_v)
        pltpu.async_copy(table_hbm.at[idx_v], rows_v, sem).wait()
        pltpu.sync_copy(rows_v, out_hbm.at[pl.ds(base, b_per_w)])

    return gather_kernel


def kernel(z_e, codebook):
    n, d = z_e.shape
    k = codebook.shape[0]
    z2 = jnp.sum(z_e ** 2, axis=1, keepdims=True)
    w2 = jnp.sum(codebook ** 2, axis=1)[None, :]
    wt2 = (2.0 * codebook).T

    grid = (n // BN,)
    idx, comm, vq = pl.pallas_call(
        _vq_body,
        grid=grid,
        in_specs=[
            pl.BlockSpec((BN, d), lambda i: (i, 0)),
            pl.BlockSpec((BN, 1), lambda i: (i, 0)),
            pl.BlockSpec((1, k), lambda i: (0, 0)),
            pl.BlockSpec((d, k), lambda i: (0, 0)),
        ],
        out_specs=[
            pl.BlockSpec((BN, 1), lambda i: (i, 0)),
            pl.BlockSpec(memory_space=pltpu.SMEM),
            pl.BlockSpec(memory_space=pltpu.SMEM),
        ],
        out_shape=[
            jax.ShapeDtypeStruct((n, 1), jnp.int32),
            jax.ShapeDtypeStruct((1, 1), jnp.float32),
            jax.ShapeDtypeStruct((1, 1), jnp.float32),
        ],
        scratch_shapes=[pltpu.SMEM((1,), jnp.float32)],
    )(z_e, z2, w2, wt2)

    idx_flat = jnp.reshape(idx, (n,))
    cb_pad = jnp.pad(codebook, ((0, 0), (0, _SC_LANES - d)))
    zq_st = _make_sc_gather(k, n, d)(cb_pad, idx_flat)[:, :d]

    return (zq_st,
            jnp.reshape(vq, ()),
            jnp.reshape(comm, ()),
            idx_flat)
